# baseline (speedup 1.0000x reference)
import functools

import jax
import jax.numpy as jnp
from jax import lax
from jax.experimental import pallas as pl
from jax.experimental.pallas import tpu as pltpu
from jax.experimental.pallas import tpu_sc as plsc

B = 16384
LOG2B = 14
F = 26
V = 1000
D = 128
NCF = 4
ROWS = B * F

SC_CORES = 2
SC_SUBCORES = 16
NW = SC_CORES * SC_SUBCORES
ROWS_PER_W = ROWS // NW
CH = 512
CHB = CH // 128
NCHUNK = ROWS_PER_W // CH


def _sc_gather(fit2, emb):
    mesh = plsc.VectorSubcoreMesh(core_axis_name="c", subcore_axis_name="s")

    @functools.partial(
        pl.kernel,
        mesh=mesh,
        out_type=jax.ShapeDtypeStruct((ROWS, D), jnp.float32),
        scratch_types=[
            pltpu.VMEM((CHB, 128), jnp.int32),
            pltpu.VMEM((CH, D), jnp.float32),
            pltpu.SemaphoreType.DMA,
        ],
    )
    def k(fi_hbm, emb_hbm, oute_hbm, idx_v, rows_v, sem_e):
        wid = lax.axis_index("c") * SC_SUBCORES + lax.axis_index("s")
        w_base = wid * ROWS_PER_W
        w_row0 = wid * (ROWS_PER_W // 128)

        def chunk_body(ci, carry):
            base = w_base + ci * CH
            rb = w_row0 + ci * CHB
            pltpu.sync_copy(fi_hbm.at[pl.ds(rb, CHB)], idx_v)
            for j in range(CHB):
                for k16 in range(8):
                    p = base + j * 128 + k16 * 16 + lax.iota(jnp.int32, 16)
                    off = lax.shift_right_logical(p, LOG2B) * V
                    sl = (j, pl.ds(k16 * 16, 16))
                    idx_v[sl] = idx_v[sl] + off
            copies = [
                pltpu.make_async_copy(
                    emb_hbm.at[idx_v.at[j]],
                    rows_v.at[pl.ds(j * 128, 128)], sem_e)
                for j in range(CHB)
            ]
            for c in copies:
                c.start()
            for c in copies:
                c.wait()
            pltpu.sync_copy(rows_v, oute_hbm.at[pl.ds(base, CH)])
            return carry

        lax.fori_loop(0, NCHUNK, chunk_body, 0)

    return k(fit2, emb)


LCH = 512
LNCHUNK = ROWS_PER_W // LCH


def _sc_linear(fi_flat, lin_flat):
    mesh = plsc.VectorSubcoreMesh(core_axis_name="c", subcore_axis_name="s")

    @functools.partial(
        pl.kernel,
        mesh=mesh,
        out_type=jax.ShapeDtypeStruct((ROWS,), jnp.float32),
        scratch_types=[
            pltpu.VMEM((F * V,), jnp.float32),
            pltpu.VMEM((LCH,), jnp.int32),
            pltpu.VMEM((LCH,), jnp.float32),
        ],
        compiler_params=pltpu.CompilerParams(needs_layout_passes=False),
    )
    def k(fi_hbm, lin_hbm, outl_hbm, tab_v, idx_v, val_v):
        wid = lax.axis_index("c") * SC_SUBCORES + lax.axis_index("s")
        w_base = wid * ROWS_PER_W
        pltpu.sync_copy(lin_hbm, tab_v)

        def chunk_body(ci, carry):
            base = w_base + ci * LCH
            pltpu.sync_copy(fi_hbm.at[pl.ds(base, LCH)], idx_v)
            for j in range(LCH // 16):
                p = base + j * 16 + lax.iota(jnp.int32, 16)
                sl = pl.ds(j * 16, 16)
                gi = idx_v[sl] + lax.shift_right_logical(p, LOG2B) * V
                val_v[sl] = plsc.load_gather(tab_v, [gi])
            pltpu.sync_copy(val_v, outl_hbm.at[pl.ds(base, LCH)])
            return carry

        lax.fori_loop(0, LNCHUNK, chunk_body, 0)

    return k(fi_flat, lin_flat)


def _ln(h, g, b):
    m = jnp.mean(h, axis=-1, keepdims=True)
    d = h - m
    v = jnp.mean(d * d, axis=-1, keepdims=True)
    return d * lax.rsqrt(v + 1e-5) * g[None, :] + b[None, :]


def _tc_body(x3_ref, cont_ref, lv_ref, w1e_ref, w1c_ref, b1_ref, g1_ref,
             be1_ref, w2_ref, b2_ref, g2_ref, be2_ref, w3_ref, b3_ref, g3_ref,
             be3_ref, w4_ref, b4_ref, out_ref, emb_ref):
    x3 = x3_ref[...]
    xf = x3[0]
    emb_ref[:, 0, :] = xf
    s = xf
    ss = xf * xf
    h = jnp.dot(xf.astype(jnp.bfloat16), w1e_ref[0:D, :],
                preferred_element_type=jnp.float32)
    for f in range(1, F):
        xf = x3[f]
        emb_ref[:, f, :] = xf
        s = s + xf
        ss = ss + xf * xf
        h = h + jnp.dot(xf.astype(jnp.bfloat16),
                        w1e_ref[f * D:(f + 1) * D, :],
                        preferred_element_type=jnp.float32)
    second = 0.5 * jnp.sum(s * s - ss, axis=1, keepdims=True)
    first = jnp.sum(lv_ref[...], axis=0)[:, None]
    h = h + jnp.dot(cont_ref[...], w1c_ref[...],
                    preferred_element_type=jnp.float32)
    h = jnp.maximum(h + b1_ref[...][None, :], 0.0)
    h = _ln(h, g1_ref[...], be1_ref[...])
    h = jnp.dot(h.astype(jnp.bfloat16), w2_ref[...],
                preferred_element_type=jnp.float32)
    h = jnp.maximum(h + b2_ref[...][None, :], 0.0)
    h = _ln(h, g2_ref[...], be2_ref[...])
    h = jnp.dot(h.astype(jnp.bfloat16), w3_ref[...],
                preferred_element_type=jnp.float32)
    h = jnp.maximum(h + b3_ref[...][None, :], 0.0)
    h = _ln(h, g3_ref[...], be3_ref[...])
    deep = jnp.dot(h, w4_ref[...], preferred_element_type=jnp.float32)
    out_ref[...] = first + second + deep + b4_ref[0]


def _tc_mlp(x3, cont, lv, w1e, w1c, b1, g1, be1, w2, b2, g2, be2, w3, b3, g3,
            be3, w4, b4):
    BB = 256
    grid = (B // BB,)
    row = lambda i: (i, 0)
    mid = lambda i: (0, i, 0)
    rep2 = lambda i: (0, 0)
    rep1 = lambda i: (0,)
    h1, h2, h3 = 1024, 512, 256
    return pl.pallas_call(
        _tc_body,
        grid=grid,
        in_specs=[
            pl.BlockSpec((F, BB, D), mid),
            pl.BlockSpec((BB, NCF), row),
            pl.BlockSpec((F, BB), lambda i: (0, i)),
            pl.BlockSpec((F * D, h1), rep2),
            pl.BlockSpec((NCF, h1), rep2),
            pl.BlockSpec((h1,), rep1),
            pl.BlockSpec((h1,), rep1),
            pl.BlockSpec((h1,), rep1),
            pl.BlockSpec((h1, h2), rep2),
            pl.BlockSpec((h2,), rep1),
            pl.BlockSpec((h2,), rep1),
            pl.BlockSpec((h2,), rep1),
            pl.BlockSpec((h2, h3), rep2),
            pl.BlockSpec((h3,), rep1),
            pl.BlockSpec((h3,), rep1),
            pl.BlockSpec((h3,), rep1),
            pl.BlockSpec((h3, 1), rep2),
            pl.BlockSpec((1,), rep1),
        ],
        out_specs=[
            pl.BlockSpec((BB, 1), row),
            pl.BlockSpec((BB, F, D), lambda i: (i, 0, 0)),
        ],
        out_shape=[
            jax.ShapeDtypeStruct((B, 1), jnp.float32),
            jax.ShapeDtypeStruct((B, F, D), jnp.float32),
        ],
        compiler_params=pltpu.CompilerParams(
            dimension_semantics=("arbitrary",)),
    )(x3, cont, lv, w1e, w1c, b1, g1, be1, w2, b2, g2, be2, w3, b3, g3, be3,
      w4, b4)


def kernel(field_indices, continuous_features, embedding, linear_emb, W1, b1,
           g1, be1, W2, b2, g2, be2, W3, b3, g3, be3, W4, b4):
    fit = field_indices.astype(jnp.int32).T
    fit2 = fit.reshape(ROWS // 128, 128)
    rows = _sc_gather(fit2, embedding)
    linvals = _sc_linear(fit.reshape(-1), linear_emb.reshape(-1))
    x3 = rows.reshape(F, B, D)
    lv = linvals.reshape(F, B)
    w1e = W1[:F * D].astype(jnp.bfloat16)
    w1c = W1[F * D:]
    logits, embeds = _tc_mlp(x3, continuous_features, lv, w1e, w1c, b1, g1,
                             be1, W2.astype(jnp.bfloat16), b2, g2, be2,
                             W3.astype(jnp.bfloat16), b3, g3, be3, W4, b4)
    return (logits, embeds)


# speedup vs baseline: 1.1319x; 1.1319x over previous
import functools

import jax
import jax.numpy as jnp
from jax import lax
from jax.experimental import pallas as pl
from jax.experimental.pallas import tpu as pltpu
from jax.experimental.pallas import tpu_sc as plsc

B = 16384
LOG2B = 14
F = 26
V = 1000
D = 128
NCF = 4
ROWS = B * F

SC_CORES = 2
SC_SUBCORES = 16
NW = SC_CORES * SC_SUBCORES
ROWS_PER_W = ROWS // NW
CH = 512
CHB = CH // 128
NCHUNK = ROWS_PER_W // CH


def _sc_gather(fit2, emb):
    mesh = plsc.VectorSubcoreMesh(core_axis_name="c", subcore_axis_name="s")

    @functools.partial(
        pl.kernel,
        mesh=mesh,
        out_type=(
            jax.ShapeDtypeStruct((ROWS, D), jnp.float32),
            jax.ShapeDtypeStruct((B, F, D), jnp.float32),
        ),
        scratch_types=[
            pltpu.VMEM((CHB, 128), jnp.int32),
            pltpu.VMEM((CH, D), jnp.float32),
            pltpu.SemaphoreType.DMA,
            pltpu.SemaphoreType.DMA,
        ],
    )
    def k(fi_hbm, emb_hbm, oute_hbm, oute3_hbm, idx_v, rows_v, sem_e, sem_w):
        wid = lax.axis_index("c") * SC_SUBCORES + lax.axis_index("s")
        w_base = wid * ROWS_PER_W
        w_row0 = wid * (ROWS_PER_W // 128)

        def chunk_body(ci, carry):
            base = w_base + ci * CH
            rb = w_row0 + ci * CHB
            fld = lax.shift_right_logical(base, LOG2B)
            s0 = base - fld * B
            pltpu.sync_copy(fi_hbm.at[pl.ds(rb, CHB)], idx_v)
            for j in range(CHB):
                for k16 in range(8):
                    p = base + j * 128 + k16 * 16 + lax.iota(jnp.int32, 16)
                    off = lax.shift_right_logical(p, LOG2B) * V
                    sl = (j, pl.ds(k16 * 16, 16))
                    idx_v[sl] = idx_v[sl] + off
            copies = [
                pltpu.make_async_copy(
                    emb_hbm.at[idx_v.at[j]],
                    rows_v.at[pl.ds(j * 128, 128)], sem_e)
                for j in range(CHB)
            ]
            for c in copies:
                c.start()
            for c in copies:
                c.wait()
            w1 = pltpu.make_async_copy(
                rows_v, oute_hbm.at[pl.ds(base, CH)], sem_w)
            w2 = pltpu.make_async_copy(
                rows_v, oute3_hbm.at[pl.ds(s0, CH), fld], sem_w)
            w1.start()
            w2.start()
            w1.wait()
            w2.wait()
            return carry

        lax.fori_loop(0, NCHUNK, chunk_body, 0)

    return k(fit2, emb)


LCH = 512
LNCHUNK = ROWS_PER_W // LCH


def _sc_linear(fi_flat, lin_flat):
    mesh = plsc.VectorSubcoreMesh(core_axis_name="c", subcore_axis_name="s")

    @functools.partial(
        pl.kernel,
        mesh=mesh,
        out_type=jax.ShapeDtypeStruct((ROWS,), jnp.float32),
        scratch_types=[
            pltpu.VMEM((F * V,), jnp.float32),
            pltpu.VMEM((LCH,), jnp.int32),
            pltpu.VMEM((LCH,), jnp.float32),
        ],
        compiler_params=pltpu.CompilerParams(needs_layout_passes=False),
    )
    def k(fi_hbm, lin_hbm, outl_hbm, tab_v, idx_v, val_v):
        wid = lax.axis_index("c") * SC_SUBCORES + lax.axis_index("s")
        w_base = wid * ROWS_PER_W
        pltpu.sync_copy(lin_hbm, tab_v)

        def chunk_body(ci, carry):
            base = w_base + ci * LCH
            pltpu.sync_copy(fi_hbm.at[pl.ds(base, LCH)], idx_v)
            for j in range(LCH // 16):
                p = base + j * 16 + lax.iota(jnp.int32, 16)
                sl = pl.ds(j * 16, 16)
                gi = idx_v[sl] + lax.shift_right_logical(p, LOG2B) * V
                val_v[sl] = plsc.load_gather(tab_v, [gi])
            pltpu.sync_copy(val_v, outl_hbm.at[pl.ds(base, LCH)])
            return carry

        lax.fori_loop(0, LNCHUNK, chunk_body, 0)

    return k(fi_flat, lin_flat)


def _ln(h, g, b):
    m = jnp.mean(h, axis=-1, keepdims=True)
    d = h - m
    v = jnp.mean(d * d, axis=-1, keepdims=True)
    return d * lax.rsqrt(v + 1e-5) * g[None, :] + b[None, :]


def _tc_body(x3_ref, cont_ref, lv_ref, w1e_ref, w1c_ref, b1_ref, g1_ref,
             be1_ref, w2_ref, b2_ref, g2_ref, be2_ref, w3_ref, b3_ref, g3_ref,
             be3_ref, w4_ref, b4_ref, out_ref):
    x3 = x3_ref[...]
    xf = x3[0]
    s = xf
    ss = xf * xf
    h = jnp.dot(xf.astype(jnp.bfloat16), w1e_ref[0:D, :],
                preferred_element_type=jnp.float32)
    for f in range(1, F):
        xf = x3[f]
        s = s + xf
        ss = ss + xf * xf
        h = h + jnp.dot(xf.astype(jnp.bfloat16),
                        w1e_ref[f * D:(f + 1) * D, :],
                        preferred_element_type=jnp.float32)
    second = 0.5 * jnp.sum(s * s - ss, axis=1, keepdims=True)
    first = jnp.sum(lv_ref[...], axis=0)[:, None]
    h = h + jnp.dot(cont_ref[...], w1c_ref[...],
                    preferred_element_type=jnp.float32)
    h = jnp.maximum(h + b1_ref[...][None, :], 0.0)
    h = _ln(h, g1_ref[...], be1_ref[...])
    h = jnp.dot(h.astype(jnp.bfloat16), w2_ref[...],
                preferred_element_type=jnp.float32)
    h = jnp.maximum(h + b2_ref[...][None, :], 0.0)
    h = _ln(h, g2_ref[...], be2_ref[...])
    h = jnp.dot(h.astype(jnp.bfloat16), w3_ref[...],
                preferred_element_type=jnp.float32)
    h = jnp.maximum(h + b3_ref[...][None, :], 0.0)
    h = _ln(h, g3_ref[...], be3_ref[...])
    deep = jnp.dot(h, w4_ref[...], preferred_element_type=jnp.float32)
    out_ref[...] = first + second + deep + b4_ref[0]


def _tc_mlp(x3, cont, lv, w1e, w1c, b1, g1, be1, w2, b2, g2, be2, w3, b3, g3,
            be3, w4, b4):
    BB = 256
    grid = (B // BB,)
    row = lambda i: (i, 0)
    mid = lambda i: (0, i, 0)
    rep2 = lambda i: (0, 0)
    rep1 = lambda i: (0,)
    h1, h2, h3 = 1024, 512, 256
    return pl.pallas_call(
        _tc_body,
        grid=grid,
        in_specs=[
            pl.BlockSpec((F, BB, D), mid),
            pl.BlockSpec((BB, NCF), row),
            pl.BlockSpec((F, BB), lambda i: (0, i)),
            pl.BlockSpec((F * D, h1), rep2),
            pl.BlockSpec((NCF, h1), rep2),
            pl.BlockSpec((h1,), rep1),
            pl.BlockSpec((h1,), rep1),
            pl.BlockSpec((h1,), rep1),
            pl.BlockSpec((h1, h2), rep2),
            pl.BlockSpec((h2,), rep1),
            pl.BlockSpec((h2,), rep1),
            pl.BlockSpec((h2,), rep1),
            pl.BlockSpec((h2, h3), rep2),
            pl.BlockSpec((h3,), rep1),
            pl.BlockSpec((h3,), rep1),
            pl.BlockSpec((h3,), rep1),
            pl.BlockSpec((h3, 1), rep2),
            pl.BlockSpec((1,), rep1),
        ],
        out_specs=pl.BlockSpec((BB, 1), row),
        out_shape=jax.ShapeDtypeStruct((B, 1), jnp.float32),
        compiler_params=pltpu.CompilerParams(
            dimension_semantics=("arbitrary",)),
    )(x3, cont, lv, w1e, w1c, b1, g1, be1, w2, b2, g2, be2, w3, b3, g3, be3,
      w4, b4)


def kernel(field_indices, continuous_features, embedding, linear_emb, W1, b1,
           g1, be1, W2, b2, g2, be2, W3, b3, g3, be3, W4, b4):
    fit = field_indices.astype(jnp.int32).T
    fit2 = fit.reshape(ROWS // 128, 128)
    rows, embeds = _sc_gather(fit2, embedding)
    linvals = _sc_linear(fit.reshape(-1), linear_emb.reshape(-1))
    x3 = rows.reshape(F, B, D)
    lv = linvals.reshape(F, B)
    w1e = W1[:F * D].astype(jnp.bfloat16)
    w1c = W1[F * D:]
    logits = _tc_mlp(x3, continuous_features, lv, w1e, w1c, b1, g1,
                     be1, W2.astype(jnp.bfloat16), b2, g2, be2,
                     W3.astype(jnp.bfloat16), b3, g3, be3, W4, b4)
    return (logits, embeds)


# speedup vs baseline: 1.1354x; 1.0031x over previous
import functools

import jax
import jax.numpy as jnp
from jax import lax
from jax.experimental import pallas as pl
from jax.experimental.pallas import tpu as pltpu
from jax.experimental.pallas import tpu_sc as plsc

B = 16384
LOG2B = 14
F = 26
V = 1000
D = 128
NCF = 4
ROWS = B * F

SC_CORES = 2
SC_SUBCORES = 16
NW = SC_CORES * SC_SUBCORES
ROWS_PER_W = ROWS // NW
CH = 512
CHB = CH // 128
NCHUNK = ROWS_PER_W // CH


def _sc_gather(fit2, emb):
    mesh = plsc.VectorSubcoreMesh(core_axis_name="c", subcore_axis_name="s")

    @functools.partial(
        pl.kernel,
        mesh=mesh,
        out_type=(
            jax.ShapeDtypeStruct((ROWS, D), jnp.float32),
            jax.ShapeDtypeStruct((B, F, D), jnp.float32),
        ),
        scratch_types=[
            pltpu.VMEM((CHB, 128), jnp.int32),
            pltpu.VMEM((CH, D), jnp.float32),
            pltpu.SemaphoreType.DMA,
            pltpu.SemaphoreType.DMA,
        ],
    )
    def k(fi_hbm, emb_hbm, oute_hbm, oute3_hbm, idx_v, rows_v, sem_e, sem_w):
        wid = lax.axis_index("c") * SC_SUBCORES + lax.axis_index("s")
        w_base = wid * ROWS_PER_W
        w_row0 = wid * (ROWS_PER_W // 128)

        def chunk_body(ci, carry):
            base = w_base + ci * CH
            rb = w_row0 + ci * CHB
            fld = lax.shift_right_logical(base, LOG2B)
            s0 = base - fld * B
            pltpu.sync_copy(fi_hbm.at[pl.ds(rb, CHB)], idx_v)
            for j in range(CHB):
                for k16 in range(8):
                    p = base + j * 128 + k16 * 16 + lax.iota(jnp.int32, 16)
                    off = lax.shift_right_logical(p, LOG2B) * V
                    sl = (j, pl.ds(k16 * 16, 16))
                    idx_v[sl] = idx_v[sl] + off
            copies = [
                pltpu.make_async_copy(
                    emb_hbm.at[idx_v.at[j]],
                    rows_v.at[pl.ds(j * 128, 128)], sem_e)
                for j in range(CHB)
            ]
            for c in copies:
                c.start()
            for c in copies:
                c.wait()
            w1 = pltpu.make_async_copy(
                rows_v, oute_hbm.at[pl.ds(base, CH)], sem_w)
            w2 = pltpu.make_async_copy(
                rows_v, oute3_hbm.at[pl.ds(s0, CH), fld], sem_w)
            w1.start()
            w2.start()
            w1.wait()
            w2.wait()
            return carry

        lax.fori_loop(0, NCHUNK, chunk_body, 0)

    return k(fit2, emb)


LCH = 512
LNCHUNK = ROWS_PER_W // LCH


def _sc_linear(fit2, lin_flat):
    mesh = plsc.VectorSubcoreMesh(core_axis_name="c", subcore_axis_name="s")

    @functools.partial(
        pl.kernel,
        mesh=mesh,
        out_type=jax.ShapeDtypeStruct((F, B), jnp.float32),
        scratch_types=[
            pltpu.VMEM((F * V,), jnp.float32),
            pltpu.VMEM((LCH,), jnp.int32),
            pltpu.VMEM((LCH,), jnp.float32),
        ],
        compiler_params=pltpu.CompilerParams(needs_layout_passes=False),
    )
    def k(fi_hbm, lin_hbm, outl_hbm, tab_v, idx_v, val_v):
        wid = lax.axis_index("c") * SC_SUBCORES + lax.axis_index("s")
        w_base = wid * ROWS_PER_W
        pltpu.sync_copy(lin_hbm, tab_v)

        def chunk_body(ci, carry):
            base = w_base + ci * LCH
            rb = base // 128
            fld = lax.shift_right_logical(base, LOG2B)
            s0 = base - fld * B
            for j in range(LCH // 128):
                pltpu.sync_copy(fi_hbm.at[rb + j],
                                idx_v.at[pl.ds(j * 128, 128)])
            for j in range(LCH // 16):
                p = base + j * 16 + lax.iota(jnp.int32, 16)
                sl = pl.ds(j * 16, 16)
                gi = idx_v[sl] + lax.shift_right_logical(p, LOG2B) * V
                val_v[sl] = plsc.load_gather(tab_v, [gi])
            pltpu.sync_copy(val_v, outl_hbm.at[fld, pl.ds(s0, LCH)])
            return carry

        lax.fori_loop(0, LNCHUNK, chunk_body, 0)

    return k(fit2, lin_flat)


def _ln(h, g, b):
    m = jnp.mean(h, axis=-1, keepdims=True)
    d = h - m
    v = jnp.mean(d * d, axis=-1, keepdims=True)
    return d * lax.rsqrt(v + 1e-5) * g[None, :] + b[None, :]


def _tc_body(x3_ref, cont_ref, lv_ref, w1e_ref, w1c_ref, b1_ref, g1_ref,
             be1_ref, w2_ref, b2_ref, g2_ref, be2_ref, w3_ref, b3_ref, g3_ref,
             be3_ref, w4_ref, b4_ref, out_ref):
    x3 = x3_ref[...]
    xf = x3[0]
    s = xf
    ss = xf * xf
    h = jnp.dot(xf.astype(jnp.bfloat16), w1e_ref[0:D, :],
                preferred_element_type=jnp.float32)
    for f in range(1, F):
        xf = x3[f]
        s = s + xf
        ss = ss + xf * xf
        h = h + jnp.dot(xf.astype(jnp.bfloat16),
                        w1e_ref[f * D:(f + 1) * D, :],
                        preferred_element_type=jnp.float32)
    second = 0.5 * jnp.sum(s * s - ss, axis=1, keepdims=True)
    first = jnp.sum(lv_ref[...], axis=0)[:, None]
    h = h + jnp.dot(cont_ref[...], w1c_ref[...],
                    preferred_element_type=jnp.float32)
    h = jnp.maximum(h + b1_ref[...][None, :], 0.0)
    h = _ln(h, g1_ref[...], be1_ref[...])
    h = jnp.dot(h.astype(jnp.bfloat16), w2_ref[...],
                preferred_element_type=jnp.float32)
    h = jnp.maximum(h + b2_ref[...][None, :], 0.0)
    h = _ln(h, g2_ref[...], be2_ref[...])
    h = jnp.dot(h.astype(jnp.bfloat16), w3_ref[...],
                preferred_element_type=jnp.float32)
    h = jnp.maximum(h + b3_ref[...][None, :], 0.0)
    h = _ln(h, g3_ref[...], be3_ref[...])
    deep = jnp.dot(h, w4_ref[...], preferred_element_type=jnp.float32)
    out_ref[...] = first + second + deep + b4_ref[0]


def _tc_mlp(x3, cont, lv, w1e, w1c, b1, g1, be1, w2, b2, g2, be2, w3, b3, g3,
            be3, w4, b4):
    BB = 512
    grid = (B // BB,)
    row = lambda i: (i, 0)
    mid = lambda i: (0, i, 0)
    rep2 = lambda i: (0, 0)
    rep1 = lambda i: (0,)
    h1, h2, h3 = 1024, 512, 256
    return pl.pallas_call(
        _tc_body,
        grid=grid,
        in_specs=[
            pl.BlockSpec((F, BB, D), mid),
            pl.BlockSpec((BB, NCF), row),
            pl.BlockSpec((F, BB), lambda i: (0, i)),
            pl.BlockSpec((F * D, h1), rep2),
            pl.BlockSpec((NCF, h1), rep2),
            pl.BlockSpec((h1,), rep1),
            pl.BlockSpec((h1,), rep1),
            pl.BlockSpec((h1,), rep1),
            pl.BlockSpec((h1, h2), rep2),
            pl.BlockSpec((h2,), rep1),
            pl.BlockSpec((h2,), rep1),
            pl.BlockSpec((h2,), rep1),
            pl.BlockSpec((h2, h3), rep2),
            pl.BlockSpec((h3,), rep1),
            pl.BlockSpec((h3,), rep1),
            pl.BlockSpec((h3,), rep1),
            pl.BlockSpec((h3, 1), rep2),
            pl.BlockSpec((1,), rep1),
        ],
        out_specs=pl.BlockSpec((BB, 1), row),
        out_shape=jax.ShapeDtypeStruct((B, 1), jnp.float32),
        compiler_params=pltpu.CompilerParams(
            dimension_semantics=("arbitrary",)),
    )(x3, cont, lv, w1e, w1c, b1, g1, be1, w2, b2, g2, be2, w3, b3, g3, be3,
      w4, b4)


def kernel(field_indices, continuous_features, embedding, linear_emb, W1, b1,
           g1, be1, W2, b2, g2, be2, W3, b3, g3, be3, W4, b4):
    fit2 = field_indices.astype(jnp.int32).T.reshape(ROWS // 128, 128)
    rows, embeds = _sc_gather(fit2, embedding)
    lv = _sc_linear(fit2, linear_emb.reshape(-1))
    x3 = rows.reshape(F, B, D)
    w1e = W1[:F * D].astype(jnp.bfloat16)
    w1c = W1[F * D:]
    logits = _tc_mlp(x3, continuous_features, lv, w1e, w1c, b1, g1,
                     be1, W2.astype(jnp.bfloat16), b2, g2, be2,
                     W3.astype(jnp.bfloat16), b3, g3, be3, W4, b4)
    return (logits, embeds)


# speedup vs baseline: 1.5955x; 1.4053x over previous
import functools

import jax
import jax.numpy as jnp
from jax import lax
from jax.experimental import pallas as pl
from jax.experimental.pallas import tpu as pltpu
from jax.experimental.pallas import tpu_sc as plsc

B = 16384
LOG2B = 14
F = 26
V = 1000
D = 128
NCF = 4
ROWS = B * F

SC_CORES = 2
SC_SUBCORES = 16
NW = SC_CORES * SC_SUBCORES
ROWS_PER_W = ROWS // NW
CH = 512
CHB = CH // 128
NCHUNK = ROWS_PER_W // CH


def _sc_gather(fit2, emb):
    mesh = plsc.VectorSubcoreMesh(core_axis_name="c", subcore_axis_name="s")

    @functools.partial(
        pl.kernel,
        mesh=mesh,
        out_type=jax.ShapeDtypeStruct((ROWS, D), jnp.float32),
        scratch_types=[
            pltpu.VMEM((CHB, 128), jnp.int32),
            pltpu.VMEM((CH, D), jnp.float32),
            pltpu.SemaphoreType.DMA,
        ],
    )
    def k(fi_hbm, emb_hbm, oute_hbm, idx_v, rows_v, sem_e):
        wid = lax.axis_index("c") * SC_SUBCORES + lax.axis_index("s")
        w_base = wid * ROWS_PER_W
        w_row0 = wid * (ROWS_PER_W // 128)

        def chunk_body(ci, carry):
            base = w_base + ci * CH
            rb = w_row0 + ci * CHB
            pltpu.sync_copy(fi_hbm.at[pl.ds(rb, CHB)], idx_v)
            for j in range(CHB):
                for k16 in range(8):
                    p = base + j * 128 + k16 * 16 + lax.iota(jnp.int32, 16)
                    off = lax.shift_right_logical(p, LOG2B) * V
                    sl = (j, pl.ds(k16 * 16, 16))
                    idx_v[sl] = idx_v[sl] + off
            copies = [
                pltpu.make_async_copy(
                    emb_hbm.at[idx_v.at[j]],
                    rows_v.at[pl.ds(j * 128, 128)], sem_e)
                for j in range(CHB)
            ]
            for c in copies:
                c.start()
            for c in copies:
                c.wait()
            pltpu.sync_copy(rows_v, oute_hbm.at[pl.ds(base, CH)])
            return carry

        lax.fori_loop(0, NCHUNK, chunk_body, 0)

    return k(fit2, emb)


LCH = 512
LNCHUNK = ROWS_PER_W // LCH


def _sc_linear(fit2, lin_flat):
    mesh = plsc.VectorSubcoreMesh(core_axis_name="c", subcore_axis_name="s")

    @functools.partial(
        pl.kernel,
        mesh=mesh,
        out_type=jax.ShapeDtypeStruct((F, B), jnp.float32),
        scratch_types=[
            pltpu.VMEM((F * V,), jnp.float32),
            pltpu.VMEM((LCH,), jnp.int32),
            pltpu.VMEM((LCH,), jnp.float32),
        ],
        compiler_params=pltpu.CompilerParams(needs_layout_passes=False),
    )
    def k(fi_hbm, lin_hbm, outl_hbm, tab_v, idx_v, val_v):
        wid = lax.axis_index("c") * SC_SUBCORES + lax.axis_index("s")
        w_base = wid * ROWS_PER_W
        pltpu.sync_copy(lin_hbm, tab_v)

        def chunk_body(ci, carry):
            base = w_base + ci * LCH
            rb = base // 128
            fld = lax.shift_right_logical(base, LOG2B)
            s0 = base - fld * B
            for j in range(LCH // 128):
                pltpu.sync_copy(fi_hbm.at[rb + j],
                                idx_v.at[pl.ds(j * 128, 128)])
            for j in range(LCH // 16):
                p = base + j * 16 + lax.iota(jnp.int32, 16)
                sl = pl.ds(j * 16, 16)
                gi = idx_v[sl] + lax.shift_right_logical(p, LOG2B) * V
                val_v[sl] = plsc.load_gather(tab_v, [gi])
            pltpu.sync_copy(val_v, outl_hbm.at[fld, pl.ds(s0, LCH)])
            return carry

        lax.fori_loop(0, LNCHUNK, chunk_body, 0)

    return k(fit2, lin_flat)


def _ln(h, g, b):
    m = jnp.mean(h, axis=-1, keepdims=True)
    d = h - m
    v = jnp.mean(d * d, axis=-1, keepdims=True)
    return d * lax.rsqrt(v + 1e-5) * g[None, :] + b[None, :]


def _tc_body(x3_ref, cont_ref, lv_ref, w1e_ref, w1c_ref, b1_ref, g1_ref,
             be1_ref, w2_ref, b2_ref, g2_ref, be2_ref, w3_ref, b3_ref, g3_ref,
             be3_ref, w4_ref, b4_ref, out_ref):
    x3 = x3_ref[...]
    xf = x3[0]
    s = xf
    ss = xf * xf
    h = jnp.dot(xf.astype(jnp.bfloat16), w1e_ref[0:D, :],
                preferred_element_type=jnp.float32)
    for f in range(1, F):
        xf = x3[f]
        s = s + xf
        ss = ss + xf * xf
        h = h + jnp.dot(xf.astype(jnp.bfloat16),
                        w1e_ref[f * D:(f + 1) * D, :],
                        preferred_element_type=jnp.float32)
    second = 0.5 * jnp.sum(s * s - ss, axis=1, keepdims=True)
    first = jnp.sum(lv_ref[...], axis=0)[:, None]
    h = h + jnp.dot(cont_ref[...], w1c_ref[...],
                    preferred_element_type=jnp.float32)
    h = jnp.maximum(h + b1_ref[...][None, :], 0.0)
    h = _ln(h, g1_ref[...], be1_ref[...])
    h = jnp.dot(h.astype(jnp.bfloat16), w2_ref[...],
                preferred_element_type=jnp.float32)
    h = jnp.maximum(h + b2_ref[...][None, :], 0.0)
    h = _ln(h, g2_ref[...], be2_ref[...])
    h = jnp.dot(h.astype(jnp.bfloat16), w3_ref[...],
                preferred_element_type=jnp.float32)
    h = jnp.maximum(h + b3_ref[...][None, :], 0.0)
    h = _ln(h, g3_ref[...], be3_ref[...])
    deep = jnp.dot(h, w4_ref[...], preferred_element_type=jnp.float32)
    out_ref[...] = first + second + deep + b4_ref[0]


def _tc_mlp(x3, cont, lv, w1e, w1c, b1, g1, be1, w2, b2, g2, be2, w3, b3, g3,
            be3, w4, b4):
    BB = 512
    grid = (B // BB,)
    row = lambda i: (i, 0)
    mid = lambda i: (0, i, 0)
    rep2 = lambda i: (0, 0)
    rep1 = lambda i: (0,)
    h1, h2, h3 = 1024, 512, 256
    return pl.pallas_call(
        _tc_body,
        grid=grid,
        in_specs=[
            pl.BlockSpec((F, BB, D), mid),
            pl.BlockSpec((BB, NCF), row),
            pl.BlockSpec((F, BB), lambda i: (0, i)),
            pl.BlockSpec((F * D, h1), rep2),
            pl.BlockSpec((NCF, h1), rep2),
            pl.BlockSpec((h1,), rep1),
            pl.BlockSpec((h1,), rep1),
            pl.BlockSpec((h1,), rep1),
            pl.BlockSpec((h1, h2), rep2),
            pl.BlockSpec((h2,), rep1),
            pl.BlockSpec((h2,), rep1),
            pl.BlockSpec((h2,), rep1),
            pl.BlockSpec((h2, h3), rep2),
            pl.BlockSpec((h3,), rep1),
            pl.BlockSpec((h3,), rep1),
            pl.BlockSpec((h3,), rep1),
            pl.BlockSpec((h3, 1), rep2),
            pl.BlockSpec((1,), rep1),
        ],
        out_specs=pl.BlockSpec((BB, 1), row),
        out_shape=jax.ShapeDtypeStruct((B, 1), jnp.float32),
        compiler_params=pltpu.CompilerParams(
            dimension_semantics=("arbitrary",)),
    )(x3, cont, lv, w1e, w1c, b1, g1, be1, w2, b2, g2, be2, w3, b3, g3, be3,
      w4, b4)


def kernel(field_indices, continuous_features, embedding, linear_emb, W1, b1,
           g1, be1, W2, b2, g2, be2, W3, b3, g3, be3, W4, b4):
    fit2 = field_indices.astype(jnp.int32).T.reshape(ROWS // 128, 128)
    rows = _sc_gather(fit2, embedding)
    lv = _sc_linear(fit2, linear_emb.reshape(-1))
    x3 = rows.reshape(F, B, D)
    embeds = x3.transpose(1, 0, 2)
    w1e = W1[:F * D].astype(jnp.bfloat16)
    w1c = W1[F * D:]
    logits = _tc_mlp(x3, continuous_features, lv, w1e, w1c, b1, g1,
                     be1, W2.astype(jnp.bfloat16), b2, g2, be2,
                     W3.astype(jnp.bfloat16), b3, g3, be3, W4, b4)
    return (logits, embeds)


# speedup vs baseline: 2.1188x; 1.3280x over previous
import functools

import jax
import jax.numpy as jnp
from jax import lax
from jax.experimental import pallas as pl
from jax.experimental.pallas import tpu as pltpu
from jax.experimental.pallas import tpu_sc as plsc

B = 16384
LOG2B = 14
F = 26
V = 1000
D = 128
NCF = 4
ROWS = B * F

SC_CORES = 2
SC_SUBCORES = 16
NW = SC_CORES * SC_SUBCORES
ROWS_PER_W = ROWS // NW
CH = 512
CHB = CH // 128
NCHUNK = ROWS_PER_W // CH


def _sc_gather(fit2, emb):
    mesh = plsc.VectorSubcoreMesh(core_axis_name="c", subcore_axis_name="s")

    @functools.partial(
        pl.kernel,
        mesh=mesh,
        out_type=jax.ShapeDtypeStruct((ROWS, D), jnp.float32),
        scratch_types=[
            pltpu.VMEM((CHB, 128), jnp.int32),
            pltpu.VMEM((CH, D), jnp.float32),
            pltpu.SemaphoreType.DMA,
        ],
    )
    def k(fi_hbm, emb_hbm, oute_hbm, idx_v, rows_v, sem_e):
        wid = lax.axis_index("c") * SC_SUBCORES + lax.axis_index("s")
        w_base = wid * ROWS_PER_W
        w_row0 = wid * (ROWS_PER_W // 128)

        def chunk_body(ci, carry):
            base = w_base + ci * CH
            rb = w_row0 + ci * CHB
            pltpu.sync_copy(fi_hbm.at[pl.ds(rb, CHB)], idx_v)
            for j in range(CHB):
                for k16 in range(8):
                    p = base + j * 128 + k16 * 16 + lax.iota(jnp.int32, 16)
                    off = lax.shift_right_logical(p, LOG2B) * V
                    sl = (j, pl.ds(k16 * 16, 16))
                    idx_v[sl] = idx_v[sl] + off
            copies = [
                pltpu.make_async_copy(
                    emb_hbm.at[idx_v.at[j]],
                    rows_v.at[pl.ds(j * 128, 128)], sem_e)
                for j in range(CHB)
            ]
            for c in copies:
                c.start()
            for c in copies:
                c.wait()
            pltpu.sync_copy(rows_v, oute_hbm.at[pl.ds(base, CH)])
            return carry

        lax.fori_loop(0, NCHUNK, chunk_body, 0)

    return k(fit2, emb)


LCH = 512
LNCHUNK = ROWS_PER_W // LCH


def _sc_linear(fi_flat, lin_flat):
    mesh = plsc.VectorSubcoreMesh(core_axis_name="c", subcore_axis_name="s")

    @functools.partial(
        pl.kernel,
        mesh=mesh,
        out_type=jax.ShapeDtypeStruct((F, B), jnp.float32),
        scratch_types=[
            pltpu.VMEM((F * V,), jnp.float32),
            pltpu.VMEM((LCH,), jnp.int32),
            pltpu.VMEM((LCH,), jnp.float32),
        ],
        compiler_params=pltpu.CompilerParams(needs_layout_passes=False),
    )
    def k(fi_hbm, lin_hbm, outl_hbm, tab_v, idx_v, val_v):
        wid = lax.axis_index("c") * SC_SUBCORES + lax.axis_index("s")
        w_base = wid * ROWS_PER_W
        pltpu.sync_copy(lin_hbm, tab_v)

        def chunk_body(ci, carry):
            base = w_base + ci * LCH
            fld = lax.shift_right_logical(base, LOG2B)
            s0 = base - fld * B
            pltpu.sync_copy(fi_hbm.at[pl.ds(base, LCH)], idx_v)
            for j in range(LCH // 16):
                p = base + j * 16 + lax.iota(jnp.int32, 16)
                sl = pl.ds(j * 16, 16)
                gi = idx_v[sl] + lax.shift_right_logical(p, LOG2B) * V
                val_v[sl] = plsc.load_gather(tab_v, [gi])
            pltpu.sync_copy(val_v, outl_hbm.at[fld, pl.ds(s0, LCH)])
            return carry

        lax.fori_loop(0, LNCHUNK, chunk_body, 0)

    return k(fi_flat, lin_flat)


def _ln(h, g, b):
    m = jnp.mean(h, axis=-1, keepdims=True)
    d = h - m
    v = jnp.mean(d * d, axis=-1, keepdims=True)
    return d * lax.rsqrt(v + 1e-5) * g[None, :] + b[None, :]


def _tc_body(x3_ref, cont_ref, lv_ref, w1e_ref, w1c_ref, b1_ref, g1_ref,
             be1_ref, w2_ref, b2_ref, g2_ref, be2_ref, w3_ref, b3_ref, g3_ref,
             be3_ref, w4_ref, b4_ref, out_ref):
    x3 = x3_ref[...]
    xf = x3[0]
    s = xf
    ss = xf * xf
    for f in range(1, F):
        xf = x3[f]
        s = s + xf
        ss = ss + xf * xf
    xall = jnp.concatenate([x3[f] for f in range(F)], axis=1)
    h = jnp.dot(xall.astype(jnp.bfloat16), w1e_ref[...],
                preferred_element_type=jnp.float32)
    second = 0.5 * jnp.sum(s * s - ss, axis=1, keepdims=True)
    first = jnp.sum(lv_ref[...], axis=0)[:, None]
    h = h + jnp.dot(cont_ref[...], w1c_ref[...],
                    preferred_element_type=jnp.float32)
    h = jnp.maximum(h + b1_ref[...][None, :], 0.0)
    h = _ln(h, g1_ref[...], be1_ref[...])
    h = jnp.dot(h.astype(jnp.bfloat16), w2_ref[...],
                preferred_element_type=jnp.float32)
    h = jnp.maximum(h + b2_ref[...][None, :], 0.0)
    h = _ln(h, g2_ref[...], be2_ref[...])
    h = jnp.dot(h.astype(jnp.bfloat16), w3_ref[...],
                preferred_element_type=jnp.float32)
    h = jnp.maximum(h + b3_ref[...][None, :], 0.0)
    h = _ln(h, g3_ref[...], be3_ref[...])
    deep = jnp.dot(h, w4_ref[...], preferred_element_type=jnp.float32)
    out_ref[...] = first + second + deep + b4_ref[0]


def _tc_mlp(x3, cont, lv, w1e, w1c, b1, g1, be1, w2, b2, g2, be2, w3, b3, g3,
            be3, w4, b4):
    BB = 512
    grid = (B // BB,)
    row = lambda i: (i, 0)
    mid = lambda i: (0, i, 0)
    rep2 = lambda i: (0, 0)
    rep1 = lambda i: (0,)
    h1, h2, h3 = 1024, 512, 256
    return pl.pallas_call(
        _tc_body,
        grid=grid,
        in_specs=[
            pl.BlockSpec((F, BB, D), mid),
            pl.BlockSpec((BB, NCF), row),
            pl.BlockSpec((F, BB), lambda i: (0, i)),
            pl.BlockSpec((F * D, h1), rep2),
            pl.BlockSpec((NCF, h1), rep2),
            pl.BlockSpec((h1,), rep1),
            pl.BlockSpec((h1,), rep1),
            pl.BlockSpec((h1,), rep1),
            pl.BlockSpec((h1, h2), rep2),
            pl.BlockSpec((h2,), rep1),
            pl.BlockSpec((h2,), rep1),
            pl.BlockSpec((h2,), rep1),
            pl.BlockSpec((h2, h3), rep2),
            pl.BlockSpec((h3,), rep1),
            pl.BlockSpec((h3,), rep1),
            pl.BlockSpec((h3,), rep1),
            pl.BlockSpec((h3, 1), rep2),
            pl.BlockSpec((1,), rep1),
        ],
        out_specs=pl.BlockSpec((BB, 1), row),
        out_shape=jax.ShapeDtypeStruct((B, 1), jnp.float32),
        compiler_params=pltpu.CompilerParams(
            dimension_semantics=("arbitrary",)),
    )(x3, cont, lv, w1e, w1c, b1, g1, be1, w2, b2, g2, be2, w3, b3, g3, be3,
      w4, b4)


def kernel(field_indices, continuous_features, embedding, linear_emb, W1, b1,
           g1, be1, W2, b2, g2, be2, W3, b3, g3, be3, W4, b4):
    fit2 = field_indices.astype(jnp.int32).T.reshape(ROWS // 128, 128)
    rows = _sc_gather(fit2, embedding)
    lv = _sc_linear(fit2.reshape(-1), linear_emb.reshape(-1))
    x3 = rows.reshape(F, B, D)
    embeds = x3.transpose(1, 0, 2)
    w1e = W1[:F * D].astype(jnp.bfloat16)
    w1c = W1[F * D:]
    logits = _tc_mlp(x3, continuous_features, lv, w1e, w1c, b1, g1,
                     be1, W2.astype(jnp.bfloat16), b2, g2, be2,
                     W3.astype(jnp.bfloat16), b3, g3, be3, W4, b4)
    return (logits, embeds)


# speedup vs baseline: 2.3034x; 1.0871x over previous
import functools

import jax
import jax.numpy as jnp
from jax import lax
from jax.experimental import pallas as pl
from jax.experimental.pallas import tpu as pltpu
from jax.experimental.pallas import tpu_sc as plsc

B = 16384
F = 26
V = 1000
D = 128
NCF = 4
ROWS = B * F
HB = B // 2
LOG2HB = 13
HROWS = HB * F

SC_CORES = 2
SC_SUBCORES = 16
NW = SC_CORES * SC_SUBCORES
HROWS_PER_W = HROWS // NW
CH = 512
CHB = CH // 128
HNCHUNK = HROWS_PER_W // CH


def _sc_gather_half(fith, emb):
    mesh = plsc.VectorSubcoreMesh(core_axis_name="c", subcore_axis_name="s")

    @functools.partial(
        pl.kernel,
        mesh=mesh,
        out_type=jax.ShapeDtypeStruct((HROWS, D), jnp.float32),
        scratch_types=[
            pltpu.VMEM((CHB, 128), jnp.int32),
            pltpu.VMEM((CH, D), jnp.float32),
            pltpu.SemaphoreType.DMA,
        ],
    )
    def k(fi_hbm, emb_hbm, oute_hbm, idx_v, rows_v, sem_e):
        wid = lax.axis_index("c") * SC_SUBCORES + lax.axis_index("s")
        w_base = wid * HROWS_PER_W

        w_row0 = wid * (HROWS_PER_W // 128)

        def chunk_body(ci, carry):
            qh0 = w_base + ci * CH
            fld = lax.shift_right_logical(qh0, LOG2HB)
            rb = w_row0 + ci * CHB
            pltpu.sync_copy(fi_hbm.at[pl.ds(rb, CHB)], idx_v)
            off = fld * V
            for j in range(CHB):
                for k16 in range(8):
                    sl = (j, pl.ds(k16 * 16, 16))
                    idx_v[sl] = idx_v[sl] + off
            copies = [
                pltpu.make_async_copy(
                    emb_hbm.at[idx_v.at[j]],
                    rows_v.at[pl.ds(j * 128, 128)], sem_e)
                for j in range(CHB)
            ]
            for c in copies:
                c.start()
            for c in copies:
                c.wait()
            pltpu.sync_copy(rows_v, oute_hbm.at[pl.ds(qh0, CH)])
            return carry

        lax.fori_loop(0, HNCHUNK, chunk_body, 0)

    return k(fith, emb)


LCH = 512
ROWS_PER_W = ROWS // NW
LNCHUNK = ROWS_PER_W // LCH
LOG2B = 14


def _sc_linear(fi_flat, lin_flat):
    mesh = plsc.VectorSubcoreMesh(core_axis_name="c", subcore_axis_name="s")

    @functools.partial(
        pl.kernel,
        mesh=mesh,
        out_type=jax.ShapeDtypeStruct((F, B), jnp.float32),
        scratch_types=[
            pltpu.VMEM((F * V,), jnp.float32),
            pltpu.VMEM((LCH,), jnp.int32),
            pltpu.VMEM((LCH,), jnp.float32),
        ],
        compiler_params=pltpu.CompilerParams(needs_layout_passes=False),
    )
    def k(fi_hbm, lin_hbm, outl_hbm, tab_v, idx_v, val_v):
        wid = lax.axis_index("c") * SC_SUBCORES + lax.axis_index("s")
        w_base = wid * ROWS_PER_W
        pltpu.sync_copy(lin_hbm, tab_v)

        def chunk_body(ci, carry):
            base = w_base + ci * LCH
            fld = lax.shift_right_logical(base, LOG2B)
            s0 = base - fld * B
            pltpu.sync_copy(fi_hbm.at[pl.ds(base, LCH)], idx_v)
            off = fld * V
            for j in range(LCH // 16):
                sl = pl.ds(j * 16, 16)
                val_v[sl] = plsc.load_gather(tab_v, [idx_v[sl] + off])
            pltpu.sync_copy(val_v, outl_hbm.at[fld, pl.ds(s0, LCH)])
            return carry

        lax.fori_loop(0, LNCHUNK, chunk_body, 0)

    return k(fi_flat, lin_flat)


def _ln(h, g, b):
    m = jnp.mean(h, axis=-1, keepdims=True)
    d = h - m
    v = jnp.mean(d * d, axis=-1, keepdims=True)
    return d * lax.rsqrt(v + 1e-5) * g[None, :] + b[None, :]


def _make_tc_body(with_prev):
    def body(x3_ref, cont_ref, lv_ref, w1e_ref, w1c_ref, b1_ref, g1_ref,
             be1_ref, w2_ref, b2_ref, g2_ref, be2_ref, w3_ref, b3_ref,
             g3_ref, be3_ref, w4_ref, b4_ref, *rest):
        if with_prev:
            _, out_ref, embout_ref = rest
        else:
            out_ref, embout_ref = rest
        x3 = x3_ref[...]
        embout_ref[...] = x3
        xf = x3[0]
        s = xf
        ss = xf * xf
        for f in range(1, F):
            xf = x3[f]
            s = s + xf
            ss = ss + xf * xf
        xall = jnp.concatenate([x3[f] for f in range(F)], axis=1)
        h = jnp.dot(xall.astype(jnp.bfloat16), w1e_ref[...],
                    preferred_element_type=jnp.float32)
        second = 0.5 * jnp.sum(s * s - ss, axis=1, keepdims=True)
        first = jnp.sum(lv_ref[...], axis=0)[:, None]
        h = h + jnp.dot(cont_ref[...], w1c_ref[...],
                        preferred_element_type=jnp.float32)
        h = jnp.maximum(h + b1_ref[...][None, :], 0.0)
        h = _ln(h, g1_ref[...], be1_ref[...])
        h = jnp.dot(h.astype(jnp.bfloat16), w2_ref[...],
                    preferred_element_type=jnp.float32)
        h = jnp.maximum(h + b2_ref[...][None, :], 0.0)
        h = _ln(h, g2_ref[...], be2_ref[...])
        h = jnp.dot(h.astype(jnp.bfloat16), w3_ref[...],
                    preferred_element_type=jnp.float32)
        h = jnp.maximum(h + b3_ref[...][None, :], 0.0)
        h = _ln(h, g3_ref[...], be3_ref[...])
        deep = jnp.dot(h, w4_ref[...], preferred_element_type=jnp.float32)
        out_ref[...] = first + second + deep + b4_ref[0]

    return body


def _tc_mlp_half(x3, cont, lv, w1e, w1c, b1, g1, be1, w2, b2, g2, be2, w3,
                 b3, g3, be3, w4, b4, half, emb_prev):
    BB = 512
    grid = (HB // BB,)
    nb0 = half * (HB // BB)
    row = lambda i: (i, 0)
    rep2 = lambda i: (0, 0)
    rep1 = lambda i: (0,)
    h1, h2, h3 = 1024, 512, 256
    in_specs = [
        pl.BlockSpec((F, BB, D), lambda i: (0, i, 0)),
        pl.BlockSpec((BB, NCF), lambda i: (nb0 + i, 0)),
        pl.BlockSpec((F, BB), lambda i: (0, nb0 + i)),
        pl.BlockSpec((F * D, h1), rep2),
        pl.BlockSpec((NCF, h1), rep2),
        pl.BlockSpec((h1,), rep1),
        pl.BlockSpec((h1,), rep1),
        pl.BlockSpec((h1,), rep1),
        pl.BlockSpec((h1, h2), rep2),
        pl.BlockSpec((h2,), rep1),
        pl.BlockSpec((h2,), rep1),
        pl.BlockSpec((h2,), rep1),
        pl.BlockSpec((h2, h3), rep2),
        pl.BlockSpec((h3,), rep1),
        pl.BlockSpec((h3,), rep1),
        pl.BlockSpec((h3,), rep1),
        pl.BlockSpec((h3, 1), rep2),
        pl.BlockSpec((1,), rep1),
    ]
    args = [x3, cont, lv, w1e, w1c, b1, g1, be1, w2, b2, g2, be2, w3, b3, g3,
            be3, w4, b4]
    aliases = {}
    if emb_prev is not None:
        in_specs.append(pl.BlockSpec(memory_space=pl.ANY))
        args.append(emb_prev)
        aliases = {18: 1}
    return pl.pallas_call(
        _make_tc_body(emb_prev is not None),
        grid=grid,
        in_specs=in_specs,
        out_specs=[
            pl.BlockSpec((BB, 1), row),
            pl.BlockSpec((F, BB, D), lambda i: (0, nb0 + i, 0)),
        ],
        out_shape=[
            jax.ShapeDtypeStruct((HB, 1), jnp.float32),
            jax.ShapeDtypeStruct((F, B, D), jnp.float32),
        ],
        input_output_aliases=aliases,
        compiler_params=pltpu.CompilerParams(
            dimension_semantics=("arbitrary",)),
    )(*args)


def kernel(field_indices, continuous_features, embedding, linear_emb, W1, b1,
           g1, be1, W2, b2, g2, be2, W3, b3, g3, be3, W4, b4):
    ft = field_indices.astype(jnp.int32).T
    fit_h0 = ft[:, :HB].reshape(HROWS // 128, 128)
    fit_h1 = ft[:, HB:].reshape(HROWS // 128, 128)
    rows0 = _sc_gather_half(fit_h0, embedding)
    lv = _sc_linear(ft.reshape(-1), linear_emb.reshape(-1))
    rows1 = _sc_gather_half(fit_h1, embedding)
    w1e = W1[:F * D].astype(jnp.bfloat16)
    w1c = W1[F * D:]
    w2b = W2.astype(jnp.bfloat16)
    w3b = W3.astype(jnp.bfloat16)
    x30 = rows0.reshape(F, HB, D)
    x31 = rows1.reshape(F, HB, D)
    l0, e0 = _tc_mlp_half(x30, continuous_features, lv, w1e, w1c, b1, g1,
                          be1, w2b, b2, g2, be2, w3b, b3, g3, be3, W4, b4,
                          0, None)
    l1, e1 = _tc_mlp_half(x31, continuous_features, lv, w1e, w1c, b1, g1,
                          be1, w2b, b2, g2, be2, w3b, b3, g3, be3, W4, b4,
                          1, e0)
    logits = jnp.concatenate([l0, l1], axis=0)
    embeds = e1.transpose(1, 0, 2)
    return (logits, embeds)
